# Initial kernel scaffold; baseline (speedup 1.0000x reference)
import jax
import jax.numpy as jnp
from jax.experimental import pallas as pl


def kernel(feats, edge_index, W1, b1, W2, b2):
    raise NotImplementedError("write your pallas kernel here")


# speedup vs baseline: 9.7022x; 9.7022x over previous
import jax
import jax.numpy as jnp
from jax import lax
from jax.experimental import pallas as pl
from jax.experimental.pallas import tpu as pltpu
from jax.experimental.pallas import tpu_sc as plsc

N = 10000
E = 320000
D = 128
DH = D // 2
NC = 2
NS = 16
NW = NC * NS
EPW = E // NW
K = 125
C = EPW // K
NP = 10112
RPT = NP // NS
DW = 16

_mesh = plsc.VectorSubcoreMesh(core_axis_name="c", subcore_axis_name="s",
                               num_cores=NC, num_subcores=NS)
_sc_params = pltpu.CompilerParams(use_tc_tiling_on_sc=False)


def _sc_agg_body(y, src_r, dst_r, z, agg_out,
                 idx_s, idx_d, rows0, rows1, acc, sem0, sem1):
    cid = lax.axis_index("c")
    sid = lax.axis_index("s")
    wid = sid * NC + cid
    row0 = sid * RPT

    pltpu.sync_copy(src_r.at[wid], idx_s)
    pltpu.sync_copy(dst_r.at[wid], idx_d)
    pltpu.sync_copy(z, acc.at[pl.ds(row0, RPT)])
    plsc.subcore_barrier()

    def gather(c, buf, sem):
        return pltpu.make_async_copy(y.at[idx_s.at[c]], buf, sem)

    gather(0, rows0, sem0).start()

    def step(i, carry):
        c = 2 * i
        gather(c + 1, rows1, sem1).start()
        gather(c, rows0, sem0).wait()
        pltpu.sync_copy(rows0, acc.at[idx_d.at[c]], add=True)

        @pl.when(c + 2 < C)
        def _():
            gather(c + 2, rows0, sem0).start()

        gather(c + 1, rows1, sem1).wait()
        pltpu.sync_copy(rows1, acc.at[idx_d.at[c + 1]], add=True)
        return carry

    lax.fori_loop(0, C // 2, step, 0)
    plsc.subcore_barrier()
    pltpu.sync_copy(acc.at[pl.ds(row0, RPT)],
                    agg_out.at[pl.ds(cid * NP + row0, RPT)])


_sc_agg = pl.kernel(
    _sc_agg_body,
    out_type=jax.ShapeDtypeStruct((NC * NP, DH), jnp.float32),
    mesh=_mesh,
    scratch_types=[
        pltpu.VMEM((C, K), jnp.int32),
        pltpu.VMEM((C, K), jnp.int32),
        pltpu.VMEM((K, DH), jnp.float32),
        pltpu.VMEM((K, DH), jnp.float32),
        pltpu.VMEM_SHARED((NP, DH), jnp.float32),
        pltpu.SemaphoreType.DMA,
        pltpu.SemaphoreType.DMA,
    ],
    compiler_params=_sc_params,
)


def _sc_deg_body(dst_r, ones, z, deg_out, idx_d, ones_v, accd):
    cid = lax.axis_index("c")
    sid = lax.axis_index("s")
    wid = sid * NC + cid
    row0 = sid * RPT

    pltpu.sync_copy(dst_r.at[wid], idx_d)
    pltpu.sync_copy(ones, ones_v)
    pltpu.sync_copy(z, accd.at[pl.ds(row0, RPT)])
    plsc.subcore_barrier()

    def step(c, carry):
        pltpu.sync_copy(ones_v, accd.at[idx_d.at[c]], add=True)
        return carry

    lax.fori_loop(0, C, step, 0)
    plsc.subcore_barrier()
    pltpu.sync_copy(accd.at[pl.ds(row0, RPT)],
                    deg_out.at[pl.ds(cid * NP + row0, RPT)])


_sc_deg = pl.kernel(
    _sc_deg_body,
    out_type=jax.ShapeDtypeStruct((NC * NP, DW), jnp.float32),
    mesh=_mesh,
    scratch_types=[
        pltpu.VMEM((C, K), jnp.int32),
        pltpu.VMEM((K, DW), jnp.float32),
        pltpu.VMEM_SHARED((NP, DW), jnp.float32),
    ],
    compiler_params=_sc_params,
)

_R = 2000


def _tc_matmul(x, w):
    def body(x_ref, w_ref, o_ref):
        o_ref[...] = jnp.dot(x_ref[...], w_ref[...],
                             preferred_element_type=jnp.float32)

    return pl.pallas_call(
        body,
        grid=(N // _R,),
        in_specs=[pl.BlockSpec((_R, D), lambda i: (i, 0)),
                  pl.BlockSpec((D, D), lambda i: (0, 0))],
        out_specs=pl.BlockSpec((_R, D), lambda i: (i, 0)),
        out_shape=jax.ShapeDtypeStruct((N, D), jnp.float32),
    )(x, w)


def _neigh(al_ref, ar_ref, y_ref, d_ref):
    a = jnp.concatenate([al_ref[0] + al_ref[1], ar_ref[0] + ar_ref[1]],
                        axis=1)
    deg = (d_ref[0] + d_ref[1]).sum(axis=-1) * (1.0 / DW)
    return (a + y_ref[...]) / (deg + 1.0)[:, None]


_agg_specs = [
    pl.BlockSpec((2, _R, DH), lambda i: (0, i, 0)),
    pl.BlockSpec((2, _R, DH), lambda i: (0, i, 0)),
    pl.BlockSpec((_R, D), lambda i: (i, 0)),
    pl.BlockSpec((2, _R, DW), lambda i: (0, i, 0)),
    pl.BlockSpec((1, D), lambda i: (0, 0)),
]


def _tc_combine_matmul(al, ar, y, degp, b, w):
    def body(al_ref, ar_ref, y_ref, d_ref, b_ref, w_ref, o_ref):
        h = jnp.maximum(_neigh(al_ref, ar_ref, y_ref, d_ref) + b_ref[...],
                        0.0)
        o_ref[...] = jnp.dot(h, w_ref[...],
                             preferred_element_type=jnp.float32)

    return pl.pallas_call(
        body,
        grid=(N // _R,),
        in_specs=_agg_specs + [pl.BlockSpec((D, D), lambda i: (0, 0))],
        out_specs=pl.BlockSpec((_R, D), lambda i: (i, 0)),
        out_shape=jax.ShapeDtypeStruct((N, D), jnp.float32),
    )(al, ar, y, degp, b, w)


def _tc_combine(al, ar, y, degp, b):
    def body(al_ref, ar_ref, y_ref, d_ref, b_ref, o_ref):
        o_ref[...] = _neigh(al_ref, ar_ref, y_ref, d_ref) + b_ref[...]

    return pl.pallas_call(
        body,
        grid=(N // _R,),
        in_specs=_agg_specs,
        out_specs=pl.BlockSpec((_R, D), lambda i: (i, 0)),
        out_shape=jax.ShapeDtypeStruct((N, D), jnp.float32),
    )(al, ar, y, degp, b)


def _layer_agg(y, src_l, src_r_idx, dst):
    yv = y.reshape(2 * N, DH)
    al = _sc_agg(yv, src_l, dst, jnp.zeros((RPT, DH), jnp.float32))
    ar = _sc_agg(yv, src_r_idx, dst, jnp.zeros((RPT, DH), jnp.float32))
    return al.reshape(NC, NP, DH), ar.reshape(NC, NP, DH)


def kernel(feats, edge_index, W1, b1, W2, b2):
    src = edge_index[0]
    dst = edge_index[1].reshape(NW, C, K)
    src_l = (2 * src).reshape(NW, C, K)
    src_r = (2 * src + 1).reshape(NW, C, K)
    b1r = b1.reshape(1, D)
    b2r = b2.reshape(1, D)

    degp = _sc_deg(dst, jnp.ones((K, DW), jnp.float32),
                   jnp.zeros((RPT, DW), jnp.float32)).reshape(NC, NP, DW)
    y1 = _tc_matmul(feats, W1)
    al1, ar1 = _layer_agg(y1, src_l, src_r, dst)
    y2 = _tc_combine_matmul(al1, ar1, y1, degp, b1r, W2)
    al2, ar2 = _layer_agg(y2, src_l, src_r, dst)
    return _tc_combine(al2, ar2, y2, degp, b2r)


# speedup vs baseline: 10.0601x; 1.0369x over previous
import jax
import jax.numpy as jnp
from jax import lax
from jax.experimental import pallas as pl
from jax.experimental.pallas import tpu as pltpu
from jax.experimental.pallas import tpu_sc as plsc

N = 10000
E = 320000
D = 128
DH = D // 2
NC = 2
NS = 16
NW = NC * NS
EPW = E // NW
K = 125
C = EPW // K
NP = 10112
RPT = NP // NS
DW = 16

_mesh = plsc.VectorSubcoreMesh(core_axis_name="c", subcore_axis_name="s",
                               num_cores=NC, num_subcores=NS)
_sc_params = pltpu.CompilerParams(use_tc_tiling_on_sc=False)


_NBUF = 4
_PF = 2


def _sc_agg_body(y, src_r, dst_r, z, agg_out, idx_s, idx_d,
                 b0, b1, b2, b3, acc, g0, g1, g2, g3, s0, s1, s2, s3):
    cid = lax.axis_index("c")
    sid = lax.axis_index("s")
    wid = sid * NC + cid
    row0 = sid * RPT
    bufs = [b0, b1, b2, b3]
    gsem = [g0, g1, g2, g3]
    ssem = [s0, s1, s2, s3]

    pltpu.sync_copy(src_r.at[wid], idx_s)
    pltpu.sync_copy(dst_r.at[wid], idx_d)
    pltpu.sync_copy(z, acc.at[pl.ds(row0, RPT)])
    plsc.subcore_barrier()

    def gather(c, j):
        return pltpu.make_async_copy(y.at[idx_s.at[c]], bufs[j], gsem[j])

    def scat_start(c, j):
        pltpu.async_copy(bufs[j], acc.at[idx_d.at[c]], ssem[j], add=True)

    def scat_wait(c, j):
        pltpu.make_async_copy(bufs[j], acc.at[idx_d.at[c]], ssem[j]).wait()

    for c in range(_PF):
        gather(c, c).start()

    def step(i, carry):
        for b in range(_NBUF):
            c = _NBUF * i + b
            gather(c, b).wait()
            scat_start(c, b)
            nb = (b + _PF) % _NBUF

            @pl.when(c + _PF < C)
            def _():
                @pl.when(c >= _PF)
                def _():
                    scat_wait(c - _PF, nb)
                gather(c + _PF, nb).start()
        return carry

    lax.fori_loop(0, C // _NBUF, step, 0)
    for c in range(C - _NBUF, C):
        scat_wait(c, c % _NBUF)
    plsc.subcore_barrier()
    pltpu.sync_copy(acc.at[pl.ds(row0, RPT)],
                    agg_out.at[pl.ds(cid * NP + row0, RPT)])


_sc_agg = pl.kernel(
    _sc_agg_body,
    out_type=jax.ShapeDtypeStruct((NC * NP, DH), jnp.float32),
    mesh=_mesh,
    scratch_types=[
        pltpu.VMEM((C, K), jnp.int32),
        pltpu.VMEM((C, K), jnp.int32),
        pltpu.VMEM((K, DH), jnp.float32),
        pltpu.VMEM((K, DH), jnp.float32),
        pltpu.VMEM((K, DH), jnp.float32),
        pltpu.VMEM((K, DH), jnp.float32),
        pltpu.VMEM_SHARED((NP, DH), jnp.float32),
        pltpu.SemaphoreType.DMA,
        pltpu.SemaphoreType.DMA,
        pltpu.SemaphoreType.DMA,
        pltpu.SemaphoreType.DMA,
        pltpu.SemaphoreType.DMA,
        pltpu.SemaphoreType.DMA,
        pltpu.SemaphoreType.DMA,
        pltpu.SemaphoreType.DMA,
    ],
    compiler_params=_sc_params,
)


def _sc_deg_body(dst_r, ones, z, deg_out, idx_d, ones_v, accd):
    cid = lax.axis_index("c")
    sid = lax.axis_index("s")
    wid = sid * NC + cid
    row0 = sid * RPT

    pltpu.sync_copy(dst_r.at[wid], idx_d)
    pltpu.sync_copy(ones, ones_v)
    pltpu.sync_copy(z, accd.at[pl.ds(row0, RPT)])
    plsc.subcore_barrier()

    def step(c, carry):
        pltpu.sync_copy(ones_v, accd.at[idx_d.at[c]], add=True)
        return carry

    lax.fori_loop(0, C, step, 0)
    plsc.subcore_barrier()
    pltpu.sync_copy(accd.at[pl.ds(row0, RPT)],
                    deg_out.at[pl.ds(cid * NP + row0, RPT)])


_sc_deg = pl.kernel(
    _sc_deg_body,
    out_type=jax.ShapeDtypeStruct((NC * NP, DW), jnp.float32),
    mesh=_mesh,
    scratch_types=[
        pltpu.VMEM((C, K), jnp.int32),
        pltpu.VMEM((K, DW), jnp.float32),
        pltpu.VMEM_SHARED((NP, DW), jnp.float32),
    ],
    compiler_params=_sc_params,
)

_R = 2000


def _tc_matmul(x, w):
    def body(x_ref, w_ref, o_ref):
        o_ref[...] = jnp.dot(x_ref[...], w_ref[...],
                             preferred_element_type=jnp.float32)

    return pl.pallas_call(
        body,
        grid=(N // _R,),
        in_specs=[pl.BlockSpec((_R, D), lambda i: (i, 0)),
                  pl.BlockSpec((D, D), lambda i: (0, 0))],
        out_specs=pl.BlockSpec((_R, D), lambda i: (i, 0)),
        out_shape=jax.ShapeDtypeStruct((N, D), jnp.float32),
    )(x, w)


def _neigh(al_ref, ar_ref, y_ref, d_ref):
    a = jnp.concatenate([al_ref[0] + al_ref[1], ar_ref[0] + ar_ref[1]],
                        axis=1)
    deg = (d_ref[0] + d_ref[1]).sum(axis=-1) * (1.0 / DW)
    return (a + y_ref[...]) / (deg + 1.0)[:, None]


_agg_specs = [
    pl.BlockSpec((2, _R, DH), lambda i: (0, i, 0)),
    pl.BlockSpec((2, _R, DH), lambda i: (0, i, 0)),
    pl.BlockSpec((_R, D), lambda i: (i, 0)),
    pl.BlockSpec((2, _R, DW), lambda i: (0, i, 0)),
    pl.BlockSpec((1, D), lambda i: (0, 0)),
]


def _tc_combine_matmul(al, ar, y, degp, b, w):
    def body(al_ref, ar_ref, y_ref, d_ref, b_ref, w_ref, o_ref):
        h = jnp.maximum(_neigh(al_ref, ar_ref, y_ref, d_ref) + b_ref[...],
                        0.0)
        o_ref[...] = jnp.dot(h, w_ref[...],
                             preferred_element_type=jnp.float32)

    return pl.pallas_call(
        body,
        grid=(N // _R,),
        in_specs=_agg_specs + [pl.BlockSpec((D, D), lambda i: (0, 0))],
        out_specs=pl.BlockSpec((_R, D), lambda i: (i, 0)),
        out_shape=jax.ShapeDtypeStruct((N, D), jnp.float32),
    )(al, ar, y, degp, b, w)


def _tc_combine(al, ar, y, degp, b):
    def body(al_ref, ar_ref, y_ref, d_ref, b_ref, o_ref):
        o_ref[...] = _neigh(al_ref, ar_ref, y_ref, d_ref) + b_ref[...]

    return pl.pallas_call(
        body,
        grid=(N // _R,),
        in_specs=_agg_specs,
        out_specs=pl.BlockSpec((_R, D), lambda i: (i, 0)),
        out_shape=jax.ShapeDtypeStruct((N, D), jnp.float32),
    )(al, ar, y, degp, b)


def _layer_agg(y, src_l, src_r_idx, dst):
    yv = y.reshape(2 * N, DH)
    al = _sc_agg(yv, src_l, dst, jnp.zeros((RPT, DH), jnp.float32))
    ar = _sc_agg(yv, src_r_idx, dst, jnp.zeros((RPT, DH), jnp.float32))
    return al.reshape(NC, NP, DH), ar.reshape(NC, NP, DH)


def kernel(feats, edge_index, W1, b1, W2, b2):
    src = edge_index[0]
    dst = edge_index[1].reshape(NW, C, K)
    src_l = (2 * src).reshape(NW, C, K)
    src_r = (2 * src + 1).reshape(NW, C, K)
    b1r = b1.reshape(1, D)
    b2r = b2.reshape(1, D)

    degp = _sc_deg(dst, jnp.ones((K, DW), jnp.float32),
                   jnp.zeros((RPT, DW), jnp.float32)).reshape(NC, NP, DW)
    y1 = _tc_matmul(feats, W1)
    al1, ar1 = _layer_agg(y1, src_l, src_r, dst)
    y2 = _tc_combine_matmul(al1, ar1, y1, degp, b1r, W2)
    al2, ar2 = _layer_agg(y2, src_l, src_r, dst)
    return _tc_combine(al2, ar2, y2, degp, b2r)


# speedup vs baseline: 10.4999x; 1.0437x over previous
import jax
import jax.numpy as jnp
from jax import lax
from jax.experimental import pallas as pl
from jax.experimental.pallas import tpu as pltpu
from jax.experimental.pallas import tpu_sc as plsc

N = 10000
E = 320000
D = 128
DH = D // 2
NC = 2
NS = 16
NW = NC * NS
EPW = E // NW
K = 125
C = EPW // K
NP = 10112
RPT = NP // NS
DW = 8
_NBUF = 4
_PF = 2

_mesh = plsc.VectorSubcoreMesh(core_axis_name="c", subcore_axis_name="s",
                               num_cores=NC, num_subcores=NS)
_sc_params = pltpu.CompilerParams(use_tc_tiling_on_sc=False)


def _sc_layer(with_deg):
    out_type = [jax.ShapeDtypeStruct((NC * NP, DH), jnp.float32),
                jax.ShapeDtypeStruct((NC * NP, DH), jnp.float32)]
    scratch = [
        pltpu.VMEM((C, K), jnp.int32),
        pltpu.VMEM((C, K), jnp.int32),
        pltpu.VMEM((C, K), jnp.int32),
        pltpu.VMEM((K, DH), jnp.float32),
        pltpu.VMEM((K, DH), jnp.float32),
        pltpu.VMEM((K, DH), jnp.float32),
        pltpu.VMEM((K, DH), jnp.float32),
        pltpu.VMEM_SHARED((NP, DH), jnp.float32),
    ] + [pltpu.SemaphoreType.DMA] * 8
    if with_deg:
        out_type.append(jax.ShapeDtypeStruct((NC * NP, DW), jnp.float32))
        scratch += [
            pltpu.VMEM((K, DW), jnp.float32),
            pltpu.VMEM_SHARED((NP, DW), jnp.float32),
        ] + [pltpu.SemaphoreType.DMA] * 4

    def body(y, srcL_r, srcR_r, dst_r, z, *rest):
        if with_deg:
            (z8, ones, aggL_out, aggR_out, deg_out,
             idx_l, idx_r, idx_d, b0, b1, b2, b3, acc,
             g0, g1, g2, g3, s0, s1, s2, s3,
             ones_v, degacc, d0, d1, d2, d3) = rest
            dsem = [d0, d1, d2, d3]
        else:
            (aggL_out, aggR_out,
             idx_l, idx_r, idx_d, b0, b1, b2, b3, acc,
             g0, g1, g2, g3, s0, s1, s2, s3) = rest

        cid = lax.axis_index("c")
        sid = lax.axis_index("s")
        wid = sid * NC + cid
        row0 = sid * RPT
        out_row = cid * NP + row0
        bufs = [b0, b1, b2, b3]
        gsem = [g0, g1, g2, g3]
        ssem = [s0, s1, s2, s3]

        pltpu.sync_copy(srcL_r.at[wid], idx_l)
        pltpu.sync_copy(srcR_r.at[wid], idx_r)
        pltpu.sync_copy(dst_r.at[wid], idx_d)
        pltpu.sync_copy(z, acc.at[pl.ds(row0, RPT)])
        if with_deg:
            pltpu.sync_copy(ones, ones_v)
            pltpu.sync_copy(z8, degacc.at[pl.ds(row0, RPT)])
        plsc.subcore_barrier()

        def gather(idx, c, j):
            return pltpu.make_async_copy(y.at[idx.at[c]], bufs[j], gsem[j])

        def scat_start(c, j):
            pltpu.async_copy(bufs[j], acc.at[idx_d.at[c]], ssem[j], add=True)

        def scat_wait(c, j):
            pltpu.make_async_copy(bufs[j], acc.at[idx_d.at[c]],
                                  ssem[j]).wait()

        def dscat_start(c, j):
            pltpu.async_copy(ones_v, degacc.at[idx_d.at[c]], dsem[j],
                             add=True)

        def dscat_wait(c, j):
            pltpu.make_async_copy(ones_v, degacc.at[idx_d.at[c]],
                                  dsem[j]).wait()

        def pass_loop(idx, do_deg):
            def step(i, carry):
                for b in range(_NBUF):
                    c = _NBUF * i + b
                    gather(idx, c, b).wait()
                    scat_start(c, b)
                    if do_deg:
                        @pl.when(c >= _NBUF)
                        def _():
                            dscat_wait(c - _NBUF, b)
                        dscat_start(c, b)
                    nb = (b + _PF) % _NBUF

                    @pl.when(c + _PF < C)
                    def _():
                        @pl.when(c >= _PF)
                        def _():
                            scat_wait(c - _PF, nb)
                        gather(idx, c + _PF, nb).start()
                return carry

            lax.fori_loop(0, C // _NBUF, step, 0)
            for c in range(C - _NBUF, C):
                scat_wait(c, c % _NBUF)
                if do_deg:
                    dscat_wait(c, c % _NBUF)

        for c in range(_PF):
            gather(idx_l, c, c).start()
        pass_loop(idx_l, with_deg)
        for c in range(_PF):
            gather(idx_r, c, c).start()
        plsc.subcore_barrier()
        pltpu.sync_copy(acc.at[pl.ds(row0, RPT)],
                        aggL_out.at[pl.ds(out_row, RPT)])
        if with_deg:
            pltpu.sync_copy(degacc.at[pl.ds(row0, RPT)],
                            deg_out.at[pl.ds(out_row, RPT)])
        pltpu.sync_copy(z, acc.at[pl.ds(row0, RPT)])
        plsc.subcore_barrier()
        pass_loop(idx_r, False)
        plsc.subcore_barrier()
        pltpu.sync_copy(acc.at[pl.ds(row0, RPT)],
                        aggR_out.at[pl.ds(out_row, RPT)])

    return pl.kernel(body, out_type=tuple(out_type), mesh=_mesh,
                     scratch_types=scratch, compiler_params=_sc_params)


_sc_layer1 = _sc_layer(with_deg=True)
_sc_layer2 = _sc_layer(with_deg=False)

_R = 2000


def _tc_matmul(x, w):
    def body(x_ref, w_ref, o_ref):
        o_ref[...] = jnp.dot(x_ref[...], w_ref[...],
                             preferred_element_type=jnp.float32)

    return pl.pallas_call(
        body,
        grid=(N // _R,),
        in_specs=[pl.BlockSpec((_R, D), lambda i: (i, 0)),
                  pl.BlockSpec((D, D), lambda i: (0, 0))],
        out_specs=pl.BlockSpec((_R, D), lambda i: (i, 0)),
        out_shape=jax.ShapeDtypeStruct((N, D), jnp.float32),
    )(x, w)


def _neigh(al_ref, ar_ref, y_ref, d_ref):
    a = jnp.concatenate([al_ref[0] + al_ref[1], ar_ref[0] + ar_ref[1]],
                        axis=1)
    deg = (d_ref[0] + d_ref[1]).sum(axis=-1) * (1.0 / DW)
    return (a + y_ref[...]) / (deg + 1.0)[:, None]


_agg_specs = [
    pl.BlockSpec((2, _R, DH), lambda i: (0, i, 0)),
    pl.BlockSpec((2, _R, DH), lambda i: (0, i, 0)),
    pl.BlockSpec((_R, D), lambda i: (i, 0)),
    pl.BlockSpec((2, _R, DW), lambda i: (0, i, 0)),
    pl.BlockSpec((1, D), lambda i: (0, 0)),
]


def _tc_combine_matmul(al, ar, y, degp, b, w):
    def body(al_ref, ar_ref, y_ref, d_ref, b_ref, w_ref, o_ref):
        h = jnp.maximum(_neigh(al_ref, ar_ref, y_ref, d_ref) + b_ref[...],
                        0.0)
        o_ref[...] = jnp.dot(h, w_ref[...],
                             preferred_element_type=jnp.float32)

    return pl.pallas_call(
        body,
        grid=(N // _R,),
        in_specs=_agg_specs + [pl.BlockSpec((D, D), lambda i: (0, 0))],
        out_specs=pl.BlockSpec((_R, D), lambda i: (i, 0)),
        out_shape=jax.ShapeDtypeStruct((N, D), jnp.float32),
    )(al, ar, y, degp, b, w)


def _tc_combine(al, ar, y, degp, b):
    def body(al_ref, ar_ref, y_ref, d_ref, b_ref, o_ref):
        o_ref[...] = _neigh(al_ref, ar_ref, y_ref, d_ref) + b_ref[...]

    return pl.pallas_call(
        body,
        grid=(N // _R,),
        in_specs=_agg_specs,
        out_specs=pl.BlockSpec((_R, D), lambda i: (i, 0)),
        out_shape=jax.ShapeDtypeStruct((N, D), jnp.float32),
    )(al, ar, y, degp, b)


def kernel(feats, edge_index, W1, b1, W2, b2):
    src = edge_index[0]
    dst = edge_index[1].reshape(NW, C, K)
    src_l = (2 * src).reshape(NW, C, K)
    src_r = (2 * src + 1).reshape(NW, C, K)
    z = jnp.zeros((RPT, DH), jnp.float32)
    z8 = jnp.zeros((RPT, DW), jnp.float32)
    ones = jnp.ones((K, DW), jnp.float32)
    b1r = b1.reshape(1, D)
    b2r = b2.reshape(1, D)

    y1 = _tc_matmul(feats, W1)
    al1, ar1, degp = _sc_layer1(y1.reshape(2 * N, DH), src_l, src_r, dst,
                                z, z8, ones)
    degp = degp.reshape(NC, NP, DW)
    y2 = _tc_combine_matmul(al1.reshape(NC, NP, DH), ar1.reshape(NC, NP, DH),
                            y1, degp, b1r, W2)
    al2, ar2 = _sc_layer2(y2.reshape(2 * N, DH), src_l, src_r, dst, z)
    return _tc_combine(al2.reshape(NC, NP, DH), ar2.reshape(NC, NP, DH),
                       y2, degp, b2r)


# speedup vs baseline: 11.7039x; 1.1147x over previous
import jax
import jax.numpy as jnp
from jax import lax
from jax.experimental import pallas as pl
from jax.experimental.pallas import tpu as pltpu
from jax.experimental.pallas import tpu_sc as plsc

N = 10000
E = 320000
D = 128
DH = D // 2
NC = 2
NS = 16
NW = NC * NS
EPW = E // NW
K = 125
C = EPW // K
NP = 10112
RPT = NP // NS
DW = 8
_NBUF = 4
_PF = 2

_mesh = plsc.VectorSubcoreMesh(core_axis_name="c", subcore_axis_name="s",
                               num_cores=NC, num_subcores=NS)
_sc_params = pltpu.CompilerParams(use_tc_tiling_on_sc=False)


def _sc_layer(with_deg):
    out_type = [jax.ShapeDtypeStruct((NC * NP, D), jnp.float32)]
    scratch = [
        pltpu.VMEM((C, K), jnp.int32),
        pltpu.VMEM((C, K), jnp.int32),
        pltpu.VMEM((C, K), jnp.int32),
        pltpu.VMEM((K, DH), jnp.float32),
        pltpu.VMEM((K, DH), jnp.float32),
        pltpu.VMEM((K, DH), jnp.float32),
        pltpu.VMEM((K, DH), jnp.float32),
        pltpu.VMEM_SHARED((NP, DH), jnp.float32),
    ] + [pltpu.SemaphoreType.DMA] * 8
    if with_deg:
        out_type.append(jax.ShapeDtypeStruct((NC * NP, DW), jnp.float32))
        scratch += [
            pltpu.VMEM((K, DW), jnp.float32),
            pltpu.VMEM_SHARED((NP, DW), jnp.float32),
        ] + [pltpu.SemaphoreType.DMA] * 4

    def body(y, srcL_r, srcR_r, dst_r, z, *rest):
        if with_deg:
            (z8, ones, agg_out, deg_out,
             idx_l, idx_r, idx_d, b0, b1, b2, b3, acc,
             g0, g1, g2, g3, s0, s1, s2, s3,
             ones_v, degacc, d0, d1, d2, d3) = rest
            dsem = [d0, d1, d2, d3]
        else:
            (agg_out,
             idx_l, idx_r, idx_d, b0, b1, b2, b3, acc,
             g0, g1, g2, g3, s0, s1, s2, s3) = rest

        cid = lax.axis_index("c")
        sid = lax.axis_index("s")
        wid = sid * NC + cid
        row0 = sid * RPT
        out_row = cid * NP + row0
        bufs = [b0, b1, b2, b3]
        gsem = [g0, g1, g2, g3]
        ssem = [s0, s1, s2, s3]

        pltpu.sync_copy(srcL_r.at[wid], idx_l)
        pltpu.sync_copy(srcR_r.at[wid], idx_r)
        pltpu.sync_copy(dst_r.at[wid], idx_d)
        pltpu.sync_copy(z, acc.at[pl.ds(row0, RPT)])
        if with_deg:
            pltpu.sync_copy(ones, ones_v)
            pltpu.sync_copy(z8, degacc.at[pl.ds(row0, RPT)])
        plsc.subcore_barrier()

        def gather(idx, c, j):
            return pltpu.make_async_copy(y.at[idx.at[c]], bufs[j], gsem[j])

        def scat_start(c, j):
            pltpu.async_copy(bufs[j], acc.at[idx_d.at[c]], ssem[j], add=True)

        def scat_wait(c, j):
            pltpu.make_async_copy(bufs[j], acc.at[idx_d.at[c]],
                                  ssem[j]).wait()

        def dscat_start(c, j):
            pltpu.async_copy(ones_v, degacc.at[idx_d.at[c]], dsem[j],
                             add=True)

        def dscat_wait(c, j):
            pltpu.make_async_copy(ones_v, degacc.at[idx_d.at[c]],
                                  dsem[j]).wait()

        def pass_loop(idx, do_deg):
            def step(i, carry):
                for b in range(_NBUF):
                    c = _NBUF * i + b
                    gather(idx, c, b).wait()
                    scat_start(c, b)
                    if do_deg:
                        @pl.when(c >= _NBUF)
                        def _():
                            dscat_wait(c - _NBUF, b)
                        dscat_start(c, b)
                    nb = (b + _PF) % _NBUF

                    @pl.when(c + _PF < C)
                    def _():
                        @pl.when(c >= _PF)
                        def _():
                            scat_wait(c - _PF, nb)
                        gather(idx, c + _PF, nb).start()
                return carry

            lax.fori_loop(0, C // _NBUF, step, 0)
            for c in range(C - _NBUF, C):
                scat_wait(c, c % _NBUF)
                if do_deg:
                    dscat_wait(c, c % _NBUF)

        for c in range(_PF):
            gather(idx_l, c, c).start()
        pass_loop(idx_l, with_deg)
        for c in range(_PF):
            gather(idx_r, c, c).start()
        plsc.subcore_barrier()
        pltpu.sync_copy(acc.at[pl.ds(row0, RPT)],
                        agg_out.at[pl.ds(out_row, RPT), pl.ds(0, DH)])
        if with_deg:
            pltpu.sync_copy(degacc.at[pl.ds(row0, RPT)],
                            deg_out.at[pl.ds(out_row, RPT)])
        pltpu.sync_copy(z, acc.at[pl.ds(row0, RPT)])
        plsc.subcore_barrier()
        pass_loop(idx_r, False)
        plsc.subcore_barrier()
        pltpu.sync_copy(acc.at[pl.ds(row0, RPT)],
                        agg_out.at[pl.ds(out_row, RPT), pl.ds(DH, DH)])

    out = tuple(out_type) if with_deg else out_type[0]
    return pl.kernel(body, out_type=out, mesh=_mesh,
                     scratch_types=scratch, compiler_params=_sc_params)


_sc_layer1 = _sc_layer(with_deg=True)
_sc_layer2 = _sc_layer(with_deg=False)

_R = 2000


def _tc_matmul(x, w):
    def body(x_ref, w_ref, o_ref):
        o_ref[...] = jnp.dot(x_ref[...], w_ref[...],
                             preferred_element_type=jnp.float32)

    return pl.pallas_call(
        body,
        grid=(N // _R,),
        in_specs=[pl.BlockSpec((_R, D), lambda i: (i, 0)),
                  pl.BlockSpec((D, D), lambda i: (0, 0))],
        out_specs=pl.BlockSpec((_R, D), lambda i: (i, 0)),
        out_shape=jax.ShapeDtypeStruct((N, D), jnp.float32),
    )(x, w)


def _neigh(a_ref, y_ref, d_ref):
    a = a_ref[0] + a_ref[1]
    deg = (d_ref[0] + d_ref[1]).sum(axis=-1) * (1.0 / DW)
    return (a + y_ref[...]) / (deg + 1.0)[:, None]


_agg_specs = [
    pl.BlockSpec((2, _R, D), lambda i: (0, i, 0)),
    pl.BlockSpec((_R, D), lambda i: (i, 0)),
    pl.BlockSpec((2, _R, DW), lambda i: (0, i, 0)),
    pl.BlockSpec((1, D), lambda i: (0, 0)),
]


def _tc_combine_matmul(a, y, degp, b, w):
    def body(a_ref, y_ref, d_ref, b_ref, w_ref, o_ref):
        h = jnp.maximum(_neigh(a_ref, y_ref, d_ref) + b_ref[...], 0.0)
        o_ref[...] = jnp.dot(h, w_ref[...],
                             preferred_element_type=jnp.float32)

    return pl.pallas_call(
        body,
        grid=(N // _R,),
        in_specs=_agg_specs + [pl.BlockSpec((D, D), lambda i: (0, 0))],
        out_specs=pl.BlockSpec((_R, D), lambda i: (i, 0)),
        out_shape=jax.ShapeDtypeStruct((N, D), jnp.float32),
    )(a, y, degp, b, w)


def _tc_combine(a, y, degp, b):
    def body(a_ref, y_ref, d_ref, b_ref, o_ref):
        o_ref[...] = _neigh(a_ref, y_ref, d_ref) + b_ref[...]

    return pl.pallas_call(
        body,
        grid=(N // _R,),
        in_specs=_agg_specs,
        out_specs=pl.BlockSpec((_R, D), lambda i: (i, 0)),
        out_shape=jax.ShapeDtypeStruct((N, D), jnp.float32),
    )(a, y, degp, b)


def kernel(feats, edge_index, W1, b1, W2, b2):
    src = edge_index[0]
    dst = edge_index[1].reshape(NW, C, K)
    src_l = (2 * src).reshape(NW, C, K)
    src_r = (2 * src + 1).reshape(NW, C, K)
    z = jnp.zeros((RPT, DH), jnp.float32)
    z8 = jnp.zeros((RPT, DW), jnp.float32)
    ones = jnp.ones((K, DW), jnp.float32)
    b1r = b1.reshape(1, D)
    b2r = b2.reshape(1, D)

    y1 = _tc_matmul(feats, W1)
    a1, degp = _sc_layer1(y1.reshape(2 * N, DH), src_l, src_r, dst,
                          z, z8, ones)
    degp = degp.reshape(NC, NP, DW)
    y2 = _tc_combine_matmul(a1.reshape(NC, NP, D), y1, degp, b1r, W2)
    a2 = _sc_layer2(y2.reshape(2 * N, DH), src_l, src_r, dst, z)
    return _tc_combine(a2.reshape(NC, NP, D), y2, degp, b2r)


# speedup vs baseline: 12.4961x; 1.0677x over previous
import jax
import jax.numpy as jnp
from jax import lax
from jax.experimental import pallas as pl
from jax.experimental.pallas import tpu as pltpu
from jax.experimental.pallas import tpu_sc as plsc

N = 10000
E = 320000
D = 128
DH = D // 2
NC = 2
NS = 16
EPS = E // NS
K = 125
C = EPS // K
NP = 10112
RPT = NP // NS
DW = 8
_NBUF = 4
_PF = 2

_mesh = plsc.VectorSubcoreMesh(core_axis_name="c", subcore_axis_name="s",
                               num_cores=NC, num_subcores=NS)
_sc_params = pltpu.CompilerParams(use_tc_tiling_on_sc=False)


def _sc_layer(with_deg):
    out_type = [jax.ShapeDtypeStruct((NP, D), jnp.float32)]
    scratch = [
        pltpu.VMEM((C, K), jnp.int32),
        pltpu.VMEM((C, K), jnp.int32),
        pltpu.VMEM((K, DH), jnp.float32),
        pltpu.VMEM((K, DH), jnp.float32),
        pltpu.VMEM((K, DH), jnp.float32),
        pltpu.VMEM((K, DH), jnp.float32),
        pltpu.VMEM_SHARED((NP, DH), jnp.float32),
    ] + [pltpu.SemaphoreType.DMA] * 8
    if with_deg:
        out_type.append(jax.ShapeDtypeStruct((NP, DW), jnp.float32))
        scratch += [
            pltpu.VMEM((K, DW), jnp.float32),
            pltpu.VMEM_SHARED((NP, DW), jnp.float32),
        ] + [pltpu.SemaphoreType.DMA] * 4

    def body(y, srcL_r, srcR_r, dst_r, z, *rest):
        if with_deg:
            (z8, ones, agg_out, deg_out,
             idx_s, idx_d, b0, b1, b2, b3, acc,
             g0, g1, g2, g3, s0, s1, s2, s3,
             ones_v, degacc, d0, d1, d2, d3) = rest
            dsem = [d0, d1, d2, d3]
        else:
            (agg_out,
             idx_s, idx_d, b0, b1, b2, b3, acc,
             g0, g1, g2, g3, s0, s1, s2, s3) = rest

        cid = lax.axis_index("c")
        sid = lax.axis_index("s")
        row0 = sid * RPT
        bufs = [b0, b1, b2, b3]
        gsem = [g0, g1, g2, g3]
        ssem = [s0, s1, s2, s3]
        is0 = cid == 0

        @pl.when(is0)
        def _():
            pltpu.sync_copy(srcL_r.at[sid], idx_s)

        @pl.when(cid == 1)
        def _():
            pltpu.sync_copy(srcR_r.at[sid], idx_s)

        pltpu.sync_copy(dst_r.at[sid], idx_d)
        pltpu.sync_copy(z, acc.at[pl.ds(row0, RPT)])
        if with_deg:
            @pl.when(is0)
            def _():
                pltpu.sync_copy(ones, ones_v)
                pltpu.sync_copy(z8, degacc.at[pl.ds(row0, RPT)])
        plsc.subcore_barrier()

        def gather(c, j):
            return pltpu.make_async_copy(y.at[idx_s.at[c]], bufs[j],
                                         gsem[j])

        def scat_start(c, j):
            pltpu.async_copy(bufs[j], acc.at[idx_d.at[c]], ssem[j],
                             add=True)

        def scat_wait(c, j):
            pltpu.make_async_copy(bufs[j], acc.at[idx_d.at[c]],
                                  ssem[j]).wait()

        def dscat_start(c, j):
            pltpu.async_copy(ones_v, degacc.at[idx_d.at[c]], dsem[j],
                             add=True)

        def dscat_wait(c, j):
            pltpu.make_async_copy(ones_v, degacc.at[idx_d.at[c]],
                                  dsem[j]).wait()

        for c in range(_PF):
            gather(c, c).start()

        def step(i, carry):
            for b in range(_NBUF):
                c = _NBUF * i + b
                gather(c, b).wait()
                scat_start(c, b)
                if with_deg:
                    @pl.when(is0)
                    def _():
                        @pl.when(c >= _NBUF)
                        def _():
                            dscat_wait(c - _NBUF, b)
                        dscat_start(c, b)
                nb = (b + _PF) % _NBUF

                @pl.when(c + _PF < C)
                def _():
                    @pl.when(c >= _PF)
                    def _():
                        scat_wait(c - _PF, nb)
                    gather(c + _PF, nb).start()
            return carry

        lax.fori_loop(0, C // _NBUF, step, 0)
        for c in range(C - _NBUF, C):
            scat_wait(c, c % _NBUF)
            if with_deg:
                @pl.when(is0)
                def _():
                    dscat_wait(c, c % _NBUF)

        plsc.subcore_barrier()
        col = cid * DH
        pltpu.sync_copy(acc.at[pl.ds(row0, RPT)],
                        agg_out.at[pl.ds(row0, RPT), pl.ds(col, DH)])
        if with_deg:
            @pl.when(is0)
            def _():
                pltpu.sync_copy(degacc.at[pl.ds(row0, RPT)],
                                deg_out.at[pl.ds(row0, RPT)])

    out = tuple(out_type) if with_deg else out_type[0]
    return pl.kernel(body, out_type=out, mesh=_mesh,
                     scratch_types=scratch, compiler_params=_sc_params)


_sc_layer1 = _sc_layer(with_deg=True)
_sc_layer2 = _sc_layer(with_deg=False)

_R = 2000


def _tc_matmul(x, w):
    def body(x_ref, w_ref, o_ref):
        o_ref[...] = jnp.dot(x_ref[...], w_ref[...],
                             preferred_element_type=jnp.float32)

    return pl.pallas_call(
        body,
        grid=(N // _R,),
        in_specs=[pl.BlockSpec((_R, D), lambda i: (i, 0)),
                  pl.BlockSpec((D, D), lambda i: (0, 0))],
        out_specs=pl.BlockSpec((_R, D), lambda i: (i, 0)),
        out_shape=jax.ShapeDtypeStruct((N, D), jnp.float32),
    )(x, w)


def _neigh(a_ref, y_ref, d_ref):
    deg = d_ref[...].sum(axis=-1) * (1.0 / DW)
    return (a_ref[...] + y_ref[...]) / (deg + 1.0)[:, None]


_agg_specs = [
    pl.BlockSpec((_R, D), lambda i: (i, 0)),
    pl.BlockSpec((_R, D), lambda i: (i, 0)),
    pl.BlockSpec((_R, DW), lambda i: (i, 0)),
    pl.BlockSpec((1, D), lambda i: (0, 0)),
]


def _tc_combine_matmul(a, y, degp, b, w):
    def body(a_ref, y_ref, d_ref, b_ref, w_ref, o_ref):
        h = jnp.maximum(_neigh(a_ref, y_ref, d_ref) + b_ref[...], 0.0)
        o_ref[...] = jnp.dot(h, w_ref[...],
                             preferred_element_type=jnp.float32)

    return pl.pallas_call(
        body,
        grid=(N // _R,),
        in_specs=_agg_specs + [pl.BlockSpec((D, D), lambda i: (0, 0))],
        out_specs=pl.BlockSpec((_R, D), lambda i: (i, 0)),
        out_shape=jax.ShapeDtypeStruct((N, D), jnp.float32),
    )(a, y, degp, b, w)


def _tc_combine(a, y, degp, b):
    def body(a_ref, y_ref, d_ref, b_ref, o_ref):
        o_ref[...] = _neigh(a_ref, y_ref, d_ref) + b_ref[...]

    return pl.pallas_call(
        body,
        grid=(N // _R,),
        in_specs=_agg_specs,
        out_specs=pl.BlockSpec((_R, D), lambda i: (i, 0)),
        out_shape=jax.ShapeDtypeStruct((N, D), jnp.float32),
    )(a, y, degp, b)


def kernel(feats, edge_index, W1, b1, W2, b2):
    src = edge_index[0]
    dst = edge_index[1].reshape(NS, C, K)
    src_l = (2 * src).reshape(NS, C, K)
    src_r = (2 * src + 1).reshape(NS, C, K)
    z = jnp.zeros((RPT, DH), jnp.float32)
    z8 = jnp.zeros((RPT, DW), jnp.float32)
    ones = jnp.ones((K, DW), jnp.float32)
    b1r = b1.reshape(1, D)
    b2r = b2.reshape(1, D)

    y1 = _tc_matmul(feats, W1)
    a1, degp = _sc_layer1(y1.reshape(2 * N, DH), src_l, src_r, dst,
                          z, z8, ones)
    y2 = _tc_combine_matmul(a1, y1, degp, b1r, W2)
    a2 = _sc_layer2(y2.reshape(2 * N, DH), src_l, src_r, dst, z)
    return _tc_combine(a2, y2, degp, b2r)


# speedup vs baseline: 12.4978x; 1.0001x over previous
import jax
import jax.numpy as jnp
from jax import lax
from jax.experimental import pallas as pl
from jax.experimental.pallas import tpu as pltpu
from jax.experimental.pallas import tpu_sc as plsc

N = 10000
E = 320000
D = 128
DH = D // 2
NC = 2
NS = 16
EPS = E // NS
K = 125
C = EPS // K
NP = 10112
RPT = NP // NS
DW = 8
_NBUF = 4
_PF = 2

_mesh = plsc.VectorSubcoreMesh(core_axis_name="c", subcore_axis_name="s",
                               num_cores=NC, num_subcores=NS)
_sc_params = pltpu.CompilerParams(use_tc_tiling_on_sc=False)


def _sc_layer(with_deg):
    out_type = [jax.ShapeDtypeStruct((NP, D), jnp.float32)]
    scratch = [
        pltpu.VMEM((C, K), jnp.int32),
        pltpu.VMEM((C, K), jnp.int32),
    ] + [pltpu.VMEM((K, DH), jnp.float32)] * _NBUF + [
        pltpu.VMEM_SHARED((NP, DH), jnp.float32),
    ] + [pltpu.SemaphoreType.DMA] * (2 * _NBUF)
    if with_deg:
        out_type.append(jax.ShapeDtypeStruct((NP, DW), jnp.float32))
        scratch += [
            pltpu.VMEM((K, DW), jnp.float32),
            pltpu.VMEM_SHARED((NP, DW), jnp.float32),
        ] + [pltpu.SemaphoreType.DMA] * _NBUF

    def body(y, srcL_r, srcR_r, dst_r, z, *rest):
        nb_ = _NBUF
        if with_deg:
            (z8, ones, agg_out, deg_out, idx_s, idx_d) = rest[:6]
            bufs = list(rest[6:6 + nb_])
            acc = rest[6 + nb_]
            gsem = list(rest[7 + nb_:7 + 2 * nb_])
            ssem = list(rest[7 + 2 * nb_:7 + 3 * nb_])
            ones_v = rest[7 + 3 * nb_]
            degacc = rest[8 + 3 * nb_]
            dsem = list(rest[9 + 3 * nb_:9 + 4 * nb_])
        else:
            (agg_out, idx_s, idx_d) = rest[:3]
            bufs = list(rest[3:3 + nb_])
            acc = rest[3 + nb_]
            gsem = list(rest[4 + nb_:4 + 2 * nb_])
            ssem = list(rest[4 + 2 * nb_:4 + 3 * nb_])

        cid = lax.axis_index("c")
        sid = lax.axis_index("s")
        row0 = sid * RPT
        is0 = cid == 0

        @pl.when(is0)
        def _():
            pltpu.sync_copy(srcL_r.at[sid], idx_s)

        @pl.when(cid == 1)
        def _():
            pltpu.sync_copy(srcR_r.at[sid], idx_s)

        pltpu.sync_copy(dst_r.at[sid], idx_d)
        pltpu.sync_copy(z, acc.at[pl.ds(row0, RPT)])
        if with_deg:
            @pl.when(is0)
            def _():
                pltpu.sync_copy(ones, ones_v)
                pltpu.sync_copy(z8, degacc.at[pl.ds(row0, RPT)])
        plsc.subcore_barrier()

        def gather(c, j):
            return pltpu.make_async_copy(y.at[idx_s.at[c]], bufs[j],
                                         gsem[j])

        def scat_start(c, j):
            pltpu.async_copy(bufs[j], acc.at[idx_d.at[c]], ssem[j],
                             add=True)

        def scat_wait(c, j):
            pltpu.make_async_copy(bufs[j], acc.at[idx_d.at[c]],
                                  ssem[j]).wait()

        def dscat_start(c, j):
            pltpu.async_copy(ones_v, degacc.at[idx_d.at[c]], dsem[j],
                             add=True)

        def dscat_wait(c, j):
            pltpu.make_async_copy(ones_v, degacc.at[idx_d.at[c]],
                                  dsem[j]).wait()

        for c in range(_PF):
            gather(c, c).start()

        def step(i, carry):
            for b in range(_NBUF):
                c = _NBUF * i + b
                gather(c, b).wait()
                scat_start(c, b)
                if with_deg:
                    @pl.when(is0)
                    def _():
                        @pl.when(c >= _NBUF)
                        def _():
                            dscat_wait(c - _NBUF, b)
                        dscat_start(c, b)
                nb = (b + _PF) % _NBUF

                @pl.when(c + _PF < C)
                def _():
                    @pl.when(c >= _PF)
                    def _():
                        scat_wait(c - _PF, nb)
                    gather(c + _PF, nb).start()
            return carry

        lax.fori_loop(0, C // _NBUF, step, 0)
        for c in range(C - _NBUF, C):
            scat_wait(c, c % _NBUF)
            if with_deg:
                @pl.when(is0)
                def _():
                    dscat_wait(c, c % _NBUF)

        plsc.subcore_barrier()
        col = cid * DH
        pltpu.sync_copy(acc.at[pl.ds(row0, RPT)],
                        agg_out.at[pl.ds(row0, RPT), pl.ds(col, DH)])
        if with_deg:
            @pl.when(is0)
            def _():
                pltpu.sync_copy(degacc.at[pl.ds(row0, RPT)],
                                deg_out.at[pl.ds(row0, RPT)])

    out = tuple(out_type) if with_deg else out_type[0]
    return pl.kernel(body, out_type=out, mesh=_mesh,
                     scratch_types=scratch, compiler_params=_sc_params)


_sc_layer1 = _sc_layer(with_deg=True)
_sc_layer2 = _sc_layer(with_deg=False)

_R = 2000


def _tc_matmul(x, w):
    def body(x_ref, w_ref, o_ref):
        o_ref[...] = jnp.dot(x_ref[...], w_ref[...],
                             preferred_element_type=jnp.float32)

    return pl.pallas_call(
        body,
        grid=(N // _R,),
        in_specs=[pl.BlockSpec((_R, D), lambda i: (i, 0)),
                  pl.BlockSpec((D, D), lambda i: (0, 0))],
        out_specs=pl.BlockSpec((_R, D), lambda i: (i, 0)),
        out_shape=jax.ShapeDtypeStruct((N, D), jnp.float32),
    )(x, w)


def _neigh(a_ref, y_ref, d_ref):
    deg = d_ref[...].sum(axis=-1) * (1.0 / DW)
    return (a_ref[...] + y_ref[...]) / (deg + 1.0)[:, None]


_agg_specs = [
    pl.BlockSpec((_R, D), lambda i: (i, 0)),
    pl.BlockSpec((_R, D), lambda i: (i, 0)),
    pl.BlockSpec((_R, DW), lambda i: (i, 0)),
    pl.BlockSpec((1, D), lambda i: (0, 0)),
]


def _tc_combine_matmul(a, y, degp, b, w):
    def body(a_ref, y_ref, d_ref, b_ref, w_ref, o_ref):
        h = jnp.maximum(_neigh(a_ref, y_ref, d_ref) + b_ref[...], 0.0)
        o_ref[...] = jnp.dot(h, w_ref[...],
                             preferred_element_type=jnp.float32)

    return pl.pallas_call(
        body,
        grid=(N // _R,),
        in_specs=_agg_specs + [pl.BlockSpec((D, D), lambda i: (0, 0))],
        out_specs=pl.BlockSpec((_R, D), lambda i: (i, 0)),
        out_shape=jax.ShapeDtypeStruct((N, D), jnp.float32),
    )(a, y, degp, b, w)


def _tc_combine(a, y, degp, b):
    def body(a_ref, y_ref, d_ref, b_ref, o_ref):
        o_ref[...] = _neigh(a_ref, y_ref, d_ref) + b_ref[...]

    return pl.pallas_call(
        body,
        grid=(N // _R,),
        in_specs=_agg_specs,
        out_specs=pl.BlockSpec((_R, D), lambda i: (i, 0)),
        out_shape=jax.ShapeDtypeStruct((N, D), jnp.float32),
    )(a, y, degp, b)


def kernel(feats, edge_index, W1, b1, W2, b2):
    src = edge_index[0]
    dst = edge_index[1].reshape(NS, C, K)
    src_l = (2 * src).reshape(NS, C, K)
    src_r = (2 * src + 1).reshape(NS, C, K)
    z = jnp.zeros((RPT, DH), jnp.float32)
    z8 = jnp.zeros((RPT, DW), jnp.float32)
    ones = jnp.ones((K, DW), jnp.float32)
    b1r = b1.reshape(1, D)
    b2r = b2.reshape(1, D)

    y1 = _tc_matmul(feats, W1)
    a1, degp = _sc_layer1(y1.reshape(2 * N, DH), src_l, src_r, dst,
                          z, z8, ones)
    y2 = _tc_combine_matmul(a1, y1, degp, b1r, W2)
    a2 = _sc_layer2(y2.reshape(2 * N, DH), src_l, src_r, dst, z)
    return _tc_combine(a2, y2, degp, b2r)


# speedup vs baseline: 12.6903x; 1.0154x over previous
import jax
import jax.numpy as jnp
from jax import lax
from jax.experimental import pallas as pl
from jax.experimental.pallas import tpu as pltpu
from jax.experimental.pallas import tpu_sc as plsc

N = 10000
E = 320000
D = 128
DH = D // 2
NC = 2
NS = 16
EPS = E // NS
K = 125
C = EPS // K
NP = 10112
RPT = NP // NS
DW = 8
_NBUF = 4
_PF = 2

_mesh = plsc.VectorSubcoreMesh(core_axis_name="c", subcore_axis_name="s",
                               num_cores=NC, num_subcores=NS)
_sc_params = pltpu.CompilerParams(use_tc_tiling_on_sc=False)


def _sc_layer(with_deg):
    out_type = [jax.ShapeDtypeStruct((NP, D), jnp.float32)]
    scratch = [
        pltpu.VMEM((C, K), jnp.int32),
        pltpu.VMEM((C, K), jnp.int32),
    ] + [pltpu.VMEM((K, DH), jnp.float32)] * _NBUF + [
        pltpu.VMEM_SHARED((NP, DH), jnp.float32),
    ] + [pltpu.SemaphoreType.DMA] * (2 * _NBUF)
    if with_deg:
        out_type.append(jax.ShapeDtypeStruct((NP, DW), jnp.float32))
        scratch += [
            pltpu.VMEM((K, DW), jnp.float32),
            pltpu.VMEM_SHARED((NP, DW), jnp.float32),
        ] + [pltpu.SemaphoreType.DMA] * _NBUF

    def body(y, srcL_r, srcR_r, dst_r, z, *rest):
        nb_ = _NBUF
        if with_deg:
            (z8, ones, agg_out, deg_out, idx_s, idx_d) = rest[:6]
            bufs = list(rest[6:6 + nb_])
            acc = rest[6 + nb_]
            gsem = list(rest[7 + nb_:7 + 2 * nb_])
            ssem = list(rest[7 + 2 * nb_:7 + 3 * nb_])
            ones_v = rest[7 + 3 * nb_]
            degacc = rest[8 + 3 * nb_]
            dsem = list(rest[9 + 3 * nb_:9 + 4 * nb_])
        else:
            (agg_out, idx_s, idx_d) = rest[:3]
            bufs = list(rest[3:3 + nb_])
            acc = rest[3 + nb_]
            gsem = list(rest[4 + nb_:4 + 2 * nb_])
            ssem = list(rest[4 + 2 * nb_:4 + 3 * nb_])

        cid = lax.axis_index("c")
        sid = lax.axis_index("s")
        row0 = sid * RPT
        is0 = cid == 0

        @pl.when(is0)
        def _():
            pltpu.sync_copy(srcL_r.at[sid], idx_s)

        @pl.when(cid == 1)
        def _():
            pltpu.sync_copy(srcR_r.at[sid], idx_s)

        pltpu.sync_copy(dst_r.at[sid], idx_d)
        pltpu.sync_copy(z, acc.at[pl.ds(row0, RPT)])
        if with_deg:
            @pl.when(is0)
            def _():
                pltpu.sync_copy(ones, ones_v)
                pltpu.sync_copy(z8, degacc.at[pl.ds(row0, RPT)])
        plsc.subcore_barrier()

        def gather(c, j):
            return pltpu.make_async_copy(y.at[idx_s.at[c]], bufs[j],
                                         gsem[j])

        def scat_start(c, j):
            pltpu.async_copy(bufs[j], acc.at[idx_d.at[c]], ssem[j],
                             add=True)

        def scat_wait(c, j):
            pltpu.make_async_copy(bufs[j], acc.at[idx_d.at[c]],
                                  ssem[j]).wait()

        def dscat_start(c, j):
            pltpu.async_copy(ones_v, degacc.at[idx_d.at[c]], dsem[j],
                             add=True)

        def dscat_wait(c, j):
            pltpu.make_async_copy(ones_v, degacc.at[idx_d.at[c]],
                                  dsem[j]).wait()

        for c in range(_PF):
            gather(c, c).start()

        def step(i, carry):
            for b in range(_NBUF):
                c = _NBUF * i + b
                gather(c, b).wait()
                scat_start(c, b)
                if with_deg:
                    @pl.when(is0)
                    def _():
                        @pl.when(c >= _NBUF)
                        def _():
                            dscat_wait(c - _NBUF, b)
                        dscat_start(c, b)
                nb = (b + _PF) % _NBUF

                @pl.when(c + _PF < C)
                def _():
                    @pl.when(c >= _PF)
                    def _():
                        scat_wait(c - _PF, nb)
                    gather(c + _PF, nb).start()
            return carry

        lax.fori_loop(0, C // _NBUF, step, 0)
        for c in range(C - _NBUF, C):
            scat_wait(c, c % _NBUF)
            if with_deg:
                @pl.when(is0)
                def _():
                    dscat_wait(c, c % _NBUF)

        plsc.subcore_barrier()
        col = cid * DH
        pltpu.sync_copy(acc.at[pl.ds(row0, RPT)],
                        agg_out.at[pl.ds(row0, RPT), pl.ds(col, DH)])
        if with_deg:
            @pl.when(is0)
            def _():
                pltpu.sync_copy(degacc.at[pl.ds(row0, RPT)],
                                deg_out.at[pl.ds(row0, RPT)])

    out = tuple(out_type) if with_deg else out_type[0]
    return pl.kernel(body, out_type=out, mesh=_mesh,
                     scratch_types=scratch, compiler_params=_sc_params)


_sc_layer1 = _sc_layer(with_deg=True)
_sc_layer2 = _sc_layer(with_deg=False)

_R = 5000


def _tc_matmul(x, w):
    def body(x_ref, w_ref, o_ref):
        o_ref[...] = jnp.dot(x_ref[...], w_ref[...],
                             preferred_element_type=jnp.float32)

    return pl.pallas_call(
        body,
        grid=(N // _R,),
        in_specs=[pl.BlockSpec((_R, D), lambda i: (i, 0)),
                  pl.BlockSpec((D, D), lambda i: (0, 0))],
        out_specs=pl.BlockSpec((_R, D), lambda i: (i, 0)),
        out_shape=jax.ShapeDtypeStruct((N, D), jnp.float32),
    )(x, w)


def _neigh(a_ref, y_ref, d_ref):
    deg = d_ref[...].sum(axis=-1) * (1.0 / DW)
    return (a_ref[...] + y_ref[...]) / (deg + 1.0)[:, None]


_agg_specs = [
    pl.BlockSpec((_R, D), lambda i: (i, 0)),
    pl.BlockSpec((_R, D), lambda i: (i, 0)),
    pl.BlockSpec((_R, DW), lambda i: (i, 0)),
    pl.BlockSpec((1, D), lambda i: (0, 0)),
]


def _tc_combine_matmul(a, y, degp, b, w):
    def body(a_ref, y_ref, d_ref, b_ref, w_ref, o_ref):
        h = jnp.maximum(_neigh(a_ref, y_ref, d_ref) + b_ref[...], 0.0)
        o_ref[...] = jnp.dot(h, w_ref[...],
                             preferred_element_type=jnp.float32)

    return pl.pallas_call(
        body,
        grid=(N // _R,),
        in_specs=_agg_specs + [pl.BlockSpec((D, D), lambda i: (0, 0))],
        out_specs=pl.BlockSpec((_R, D), lambda i: (i, 0)),
        out_shape=jax.ShapeDtypeStruct((N, D), jnp.float32),
    )(a, y, degp, b, w)


def _tc_combine(a, y, degp, b):
    def body(a_ref, y_ref, d_ref, b_ref, o_ref):
        o_ref[...] = _neigh(a_ref, y_ref, d_ref) + b_ref[...]

    return pl.pallas_call(
        body,
        grid=(N // _R,),
        in_specs=_agg_specs,
        out_specs=pl.BlockSpec((_R, D), lambda i: (i, 0)),
        out_shape=jax.ShapeDtypeStruct((N, D), jnp.float32),
    )(a, y, degp, b)


def kernel(feats, edge_index, W1, b1, W2, b2):
    src = edge_index[0]
    dst = edge_index[1].reshape(NS, C, K)
    src_l = (2 * src).reshape(NS, C, K)
    src_r = (2 * src + 1).reshape(NS, C, K)
    z = jnp.zeros((RPT, DH), jnp.float32)
    z8 = jnp.zeros((RPT, DW), jnp.float32)
    ones = jnp.ones((K, DW), jnp.float32)
    b1r = b1.reshape(1, D)
    b2r = b2.reshape(1, D)

    y1 = _tc_matmul(feats, W1)
    a1, degp = _sc_layer1(y1.reshape(2 * N, DH), src_l, src_r, dst,
                          z, z8, ones)
    y2 = _tc_combine_matmul(a1, y1, degp, b1r, W2)
    a2 = _sc_layer2(y2.reshape(2 * N, DH), src_l, src_r, dst, z)
    return _tc_combine(a2, y2, degp, b2r)
